# Initial kernel scaffold; baseline (speedup 1.0000x reference)
import jax
import jax.numpy as jnp
from jax.experimental import pallas as pl


def kernel(input):
    raise NotImplementedError("write your pallas kernel here")


# speedup vs baseline: 20.7714x; 20.7714x over previous
import jax
import jax.numpy as jnp
from jax.experimental import pallas as pl
from jax.experimental.pallas import tpu as pltpu

_N = 8192
_D = 4096
_BLOCK_ROWS = 256


def _body(x_ref, o_ref):
    x = x_ref[...]
    r, d = x.shape
    p = jax.lax.broadcasted_iota(jnp.int32, (r, d), 1) & 3
    right1 = pltpu.roll(x, d - 1, 1)
    left1 = pltpu.roll(x, 1, 1)
    s1 = jnp.where((p & 1) == 0, right1, left1)
    mx = jnp.maximum(x, s1)
    mn = jnp.minimum(x, s1)
    mx_sw = jnp.where(p < 2, pltpu.roll(mx, d - 2, 1), pltpu.roll(mx, 2, 1))
    mn_sw = jnp.where(p < 2, pltpu.roll(mn, d - 2, 1), pltpu.roll(mn, 2, 1))
    second = jnp.maximum(jnp.minimum(mx, mx_sw), jnp.maximum(mn, mn_sw))
    o_ref[...] = jnp.where(x >= second, x, jnp.zeros_like(x))


def kernel(input):
    n, d = input.shape
    grid = n // _BLOCK_ROWS
    return pl.pallas_call(
        _body,
        grid=(grid,),
        in_specs=[pl.BlockSpec((_BLOCK_ROWS, d), lambda i: (i, 0))],
        out_specs=pl.BlockSpec((_BLOCK_ROWS, d), lambda i: (i, 0)),
        out_shape=jax.ShapeDtypeStruct((n, d), input.dtype),
        compiler_params=pltpu.CompilerParams(
            dimension_semantics=("arbitrary",),
        ),
    )(input)
